# baseline (speedup 1.0000x reference)
import jax
import jax.numpy as jnp
from jax.experimental import pallas as pl
from jax.experimental.pallas import tpu as pltpu
from jax.experimental.pallas import tpu_sc as plsc

_BB = 128
_BA = 16
_SCW = 128


def _tc_body(s_ref, y_ref, w_ref, o_ref):
    j = pl.program_id(1)
    nn, f = y_ref.shape[1], y_ref.shape[2]
    fs = 128
    fc_w = 32
    s = s_ref[...]
    main = jnp.zeros((_BB,), jnp.float32)
    for f0 in range(0, f, fc_w):
        acc = jnp.zeros((fc_w, _BB), jnp.float32)
        for i in range(nn):
            w0 = w_ref[0, i, f0:f0 + fc_w]
            w1 = w_ref[1, i, f0:f0 + fc_w]
            if f0 < fs:
                w2 = w_ref[2, i, f0:f0 + fc_w]
                w3 = w_ref[3, i, f0:f0 + fc_w]
            for a in range(_BA):
                sa = s[a:a + 1, :]
                wlo = jnp.where(sa == 1, w1, w0)
                if f0 < fs:
                    whi = jnp.where(sa == 3, w3, w2)
                    wsel = jnp.where(sa >= 2, whi, wlo)
                else:
                    wsel = jnp.where(sa >= 2, 0.0, wlo)
                acc = acc + y_ref[a, i, f0:f0 + fc_w] * wsel
        main = main + jnp.sum(acc, axis=0)
    val = main.reshape(1, 1, _BB)

    @pl.when(j == 0)
    def _init():
        o_ref[...] = val

    @pl.when(j > 0)
    def _accum():
        o_ref[...] = o_ref[...] + val


def _sc_sae(s_t, c_tab, b):
    a = s_t.shape[0]
    mesh = plsc.VectorSubcoreMesh(core_axis_name="c", subcore_axis_name="s")

    @pl.kernel(out_type=jax.ShapeDtypeStruct((1, b), jnp.float32), mesh=mesh)
    def sc_kernel(s_hbm, c_hbm, o_hbm):
        def body(s_vmem, c_vmem, o_vmem):
            c0 = c_vmem[0:1, 0:1]
            c1 = c_vmem[0:1, 1:2]
            c2 = c_vmem[0:1, 2:3]
            c3 = c_vmem[0:1, 3:4]
            acc = jnp.zeros((1, _SCW), jnp.float32)
            for at in range(a):
                sv = s_vmem[at:at + 1, :]
                clo = jnp.where(sv == 1, c1, c0)
                chi = jnp.where(sv == 3, c3, c2)
                acc = acc + jnp.where(sv >= 2, chi, clo)
            o_vmem[...] = acc

        pltpu.emit_pipeline(
            body,
            grid=(b // _SCW,),
            in_specs=[
                pl.BlockSpec((a, _SCW), lambda i: (0, i)),
                pl.BlockSpec((1, 4), lambda i: (0, 0)),
            ],
            out_specs=[pl.BlockSpec((1, _SCW), lambda i: (0, i))],
            core_axis_name=("c", "s"),
            dimension_semantics=(pltpu.PARALLEL,),
        )(s_hbm, c_hbm, o_hbm)

    return sc_kernel(s_t, c_tab)


def kernel(species, y, W_big, b_big, W_small, b_small, self_energies):
    b, a, nn, f = y.shape
    fs = W_small.shape[-1]
    inv = 1.0 / nn
    wb = jnp.transpose(W_big, (1, 0, 2))
    ws = jnp.pad(jnp.transpose(W_small, (1, 0, 2)),
                 ((0, 0), (0, 0), (0, f - fs)))
    w_tab = (jnp.concatenate([wb, ws], axis=0) * inv).astype(jnp.float32)
    w_bcast = jnp.broadcast_to(w_tab[:, :, :, None], (4, nn, f, _BB))
    c_tab = (jnp.concatenate([jnp.sum(b_big, 0), jnp.sum(b_small, 0)], 0) * inv
             + self_energies).reshape(1, 4).astype(jnp.float32)
    y_t = jnp.transpose(y, (1, 2, 3, 0))
    s_t = jnp.transpose(species.astype(jnp.int32), (1, 0))

    tc_out = pl.pallas_call(
        _tc_body,
        grid=(b // _BB, a // _BA),
        in_specs=[
            pl.BlockSpec((_BA, _BB), lambda i, j: (j, i)),
            pl.BlockSpec((_BA, nn, f, _BB), lambda i, j: (j, 0, 0, i)),
            pl.BlockSpec((4, nn, f, _BB), lambda i, j: (0, 0, 0, 0)),
        ],
        out_specs=pl.BlockSpec((1, 1, _BB), lambda i, j: (i, 0, 0)),
        out_shape=jax.ShapeDtypeStruct((b // _BB, 1, _BB), jnp.float32),
    )(s_t, y_t, w_bcast)
    sc_out = _sc_sae(s_t, c_tab, b)
    return tc_out.reshape(b) + sc_out.reshape(b)


# speedup vs baseline: 1.1459x; 1.1459x over previous
import jax
import jax.numpy as jnp
from jax.experimental import pallas as pl

_BB = 128
_BA = 16


def _tc_body(s_ref, y_ref, w_ref, c_ref, o_ref):
    j = pl.program_id(1)
    nn, f = y_ref.shape[1], y_ref.shape[2]
    fs = 128
    fc_w = 40
    s = s_ref[...]
    main = jnp.zeros((_BB,), jnp.float32)
    for f0 in range(0, f, fc_w):
        acc = jnp.zeros((fc_w, _BB), jnp.float32)
        for i in range(nn):
            w0 = w_ref[0, i, f0:f0 + fc_w]
            w1 = w_ref[1, i, f0:f0 + fc_w]
            if f0 < fs:
                w2 = w_ref[2, i, f0:f0 + fc_w]
                w3 = w_ref[3, i, f0:f0 + fc_w]
            for a in range(_BA):
                sa = s[a:a + 1, :]
                wlo = jnp.where(sa == 1, w1, w0)
                if f0 < fs:
                    whi = jnp.where(sa == 3, w3, w2)
                    wsel = jnp.where(sa >= 2, whi, wlo)
                else:
                    wsel = jnp.where(sa >= 2, 0.0, wlo)
                acc = acc + y_ref[a, i, f0:f0 + fc_w] * wsel
        main = main + jnp.sum(acc, axis=0)
    c = c_ref[...]
    c01 = jnp.where(s == 1, c[1:2, 0:1], c[0:1, 0:1])
    c23 = jnp.where(s == 3, c[3:4, 0:1], c[2:3, 0:1])
    ca = jnp.where(s >= 2, c23, c01)
    val = (main + jnp.sum(ca, axis=0)).reshape(1, 1, _BB)

    @pl.when(j == 0)
    def _init():
        o_ref[...] = val

    @pl.when(j > 0)
    def _accum():
        o_ref[...] = o_ref[...] + val


def kernel(species, y, W_big, b_big, W_small, b_small, self_energies):
    b, a, nn, f = y.shape
    fs = W_small.shape[-1]
    inv = 1.0 / nn
    wb = jnp.transpose(W_big, (1, 0, 2))
    ws = jnp.pad(jnp.transpose(W_small, (1, 0, 2)),
                 ((0, 0), (0, 0), (0, f - fs)))
    w_tab = (jnp.concatenate([wb, ws], axis=0) * inv).astype(jnp.float32)
    w_bcast = jnp.broadcast_to(w_tab[:, :, :, None], (4, nn, f, _BB))
    c_tab = (jnp.concatenate([jnp.sum(b_big, 0), jnp.sum(b_small, 0)], 0) * inv
             + self_energies).reshape(4, 1).astype(jnp.float32)
    y_t = jnp.transpose(y, (1, 2, 3, 0))
    s_t = jnp.transpose(species.astype(jnp.int32), (1, 0))

    out = pl.pallas_call(
        _tc_body,
        grid=(b // _BB, a // _BA),
        in_specs=[
            pl.BlockSpec((_BA, _BB), lambda i, j: (j, i)),
            pl.BlockSpec((_BA, nn, f, _BB), lambda i, j: (j, 0, 0, i)),
            pl.BlockSpec((4, nn, f, _BB), lambda i, j: (0, 0, 0, 0)),
            pl.BlockSpec((4, 1), lambda i, j: (0, 0)),
        ],
        out_specs=pl.BlockSpec((1, 1, _BB), lambda i, j: (i, 0, 0)),
        out_shape=jax.ShapeDtypeStruct((b // _BB, 1, _BB), jnp.float32),
    )(s_t, y_t, w_bcast, c_tab)
    return out.reshape(b)
